# baseline (speedup 1.0000x reference)
import functools

import jax
import jax.numpy as jnp
from jax import lax
from jax.experimental import pallas as pl
from jax.experimental.pallas import tpu as pltpu
from jax.experimental.pallas import tpu_sc as plsc

N = 10000
E = 320000
D = 128
NPAD = 10240
NW = 32
EPW = E // NW
CH = 80
NCHUNK = 128
NSUB = NPAD // 16

_mesh = plsc.VectorSubcoreMesh(core_axis_name="c", subcore_axis_name="s")


def _worker_id():
    return lax.axis_index("s") * 2 + lax.axis_index("c")


@functools.partial(
    pl.kernel,
    out_type=(
        jax.ShapeDtypeStruct((2, NPAD), jnp.float32),
        jax.ShapeDtypeStruct((NW, NCHUNK, CH), jnp.float32),
    ),
    mesh=_mesh,
    scratch_types=[
        pltpu.VMEM((NCHUNK, CH), jnp.int32),
        pltpu.VMEM((NCHUNK, CH), jnp.int32),
        pltpu.VMEM((NCHUNK, CH), jnp.float32),
        pltpu.VMEM((NCHUNK, CH), jnp.float32),
        pltpu.VMEM((NSUB,), jnp.float32),
        pltpu.VMEM_SHARED((NPAD,), jnp.float32),
    ],
)
def _deg(row3, col3, w3, out, wm_out, rowv, colv, wv, wmv, zbuf, dacc):
    cid = lax.axis_index("c")
    sid = lax.axis_index("s")
    wid = _worker_id()
    pltpu.sync_copy(row3.at[wid], rowv)
    pltpu.sync_copy(col3.at[wid], colv)
    pltpu.sync_copy(w3.at[wid], wv)

    def zb(k, _):
        zbuf[pl.ds(k * 16, 16)] = jnp.zeros((16,), jnp.float32)
        return 0

    lax.fori_loop(0, NSUB // 16, zb, 0)
    pltpu.sync_copy(zbuf, dacc.at[pl.ds(sid * NSUB, NSUB)])
    plsc.subcore_barrier()

    def cm(t, _):
        j = t // (CH // 16)
        k = t % (CH // 16)
        sl = pl.ds(k * 16, 16)
        r = rowv[j, sl]
        c = colv[j, sl]
        w = wv[j, sl]
        wmv[j, sl] = jnp.where(r != c, w, 0.0)
        return 0

    lax.fori_loop(0, NCHUNK * (CH // 16), cm, 0)
    pltpu.sync_copy(wmv, wm_out.at[wid])

    def sc(j, _):
        pltpu.sync_copy(wmv.at[j], dacc.at[rowv.at[j]], add=True)
        return 0

    lax.fori_loop(0, NCHUNK, sc, 0)
    plsc.subcore_barrier()
    pltpu.sync_copy(dacc.at[pl.ds(sid * NSUB, NSUB)],
                    out.at[cid, pl.ds(sid * NSUB, NSUB)])


GB = 8
NGRP = NCHUNK // GB


@functools.partial(
    pl.kernel,
    out_type=jax.ShapeDtypeStruct((2, NPAD, D), jnp.float32),
    mesh=_mesh,
    scratch_types=[
        pltpu.VMEM((GB, CH), jnp.int32),
        pltpu.VMEM((GB, CH), jnp.int32),
        pltpu.VMEM((GB, CH), jnp.float32),
        pltpu.VMEM((2, CH, D // 2), jnp.int32),
        pltpu.VMEM((2, CH, D), jnp.float32),
        pltpu.VMEM((2 * CH,), jnp.int32),
        pltpu.VMEM((2, CH), jnp.int32),
        pltpu.VMEM_SHARED((NPAD, D), jnp.float32),
        pltpu.SemaphoreType.DMA,
        pltpu.SemaphoreType.DMA,
        pltpu.SemaphoreType.DMA,
        pltpu.SemaphoreType.DMA,
    ],
    compiler_params=pltpu.CompilerParams(needs_layout_passes=False,
                                         use_tc_tiling_on_sc=False),
)
def _prop(u, row3, col3, wm3, out, rowg, colg, wg, rows_g, rows_s,
          grow, scol, acc, gsem0, gsem1, ssem0, ssem1):
    cid = lax.axis_index("c")
    sid = lax.axis_index("s")
    wid = _worker_id()
    gsem = (gsem0, gsem1)
    ssem = (ssem0, ssem1)

    def zb(t, _):
        i = t // (D // 16)
        k = t % (D // 16)
        rows_s[0, i, pl.ds(k * 16, 16)] = jnp.zeros((16,), jnp.float32)
        return 0

    lax.fori_loop(0, CH * (D // 16), zb, 0)

    pltpu.sync_copy(row3.at[wid, pl.ds(0, GB)], rowg)
    pltpu.sync_copy(col3.at[wid, pl.ds(0, GB)], colg)
    pltpu.sync_copy(wm3.at[wid, pl.ds(0, GB)], wg)
    for b in range(2):
        for q in range(CH // 16):
            grow[pl.ds(b * CH + q * 16, 16)] = rowg[b, pl.ds(q * 16, 16)]
    pltpu.async_copy(u.at[grow.at[pl.ds(0, CH)]], rows_g.at[0], gsem[0])
    pltpu.async_copy(u.at[grow.at[pl.ds(CH, CH)]], rows_g.at[1], gsem[1])

    def zc(q, _):
        pltpu.sync_copy(rows_s.at[0], acc.at[pl.ds(sid * NSUB + q * CH, CH)])
        return 0

    lax.fori_loop(0, NSUB // CH, zc, 0)
    plsc.subcore_barrier()

    def pair(s, _):
        for b in range(2):
            t = 2 * s + b
            jj = lax.rem(t, GB)

            if b == 0:
                @pl.when(jnp.logical_and(lax.rem(s, GB // 2) == 0, s > 0))
                def _():
                    sl_g = pl.ds((2 * s // GB) * GB, GB)
                    pltpu.sync_copy(row3.at[wid, sl_g], rowg)
                    pltpu.sync_copy(col3.at[wid, sl_g], colg)
                    pltpu.sync_copy(wm3.at[wid, sl_g], wg)

            pltpu.make_async_copy(
                u.at[grow.at[pl.ds(b * CH, CH)]], rows_g.at[b], gsem[b]).wait()

            @pl.when(s > 0)
            def _():
                pltpu.make_async_copy(
                    rows_s.at[b], acc.at[scol.at[b]], ssem[b]).wait()

            for q in range(CH // 16):
                scol[b, pl.ds(q * 16, 16)] = colg[jj, pl.ds(q * 16, 16)]

            def srow(q, _):
                wvec = wg[jj, pl.ds(q * 16, 16)]
                base = q * 16
                for l in range(16):
                    sc = wvec[l]
                    for k in range(D // 32):
                        v = rows_g[b, base + l, pl.ds(k * 16, 16)]
                        lo = plsc.bitcast(v << 16, jnp.float32)
                        hi = plsc.bitcast(v & jnp.int32(-65536), jnp.float32)
                        rows_s[b, base + l, pl.ds(k * 32, 16)] = lo * sc
                        rows_s[b, base + l, pl.ds(k * 32 + 16, 16)] = hi * sc
                return 0

            lax.fori_loop(0, CH // 16, srow, 0)

            pltpu.async_copy(
                rows_s.at[b], acc.at[scol.at[b]], ssem[b], add=True)

            @pl.when(t < NCHUNK - 2)
            def _():
                jj2 = lax.rem(t + 2, GB)

                @pl.when(jj2 >= 2)
                def _():
                    for q in range(CH // 16):
                        grow[pl.ds(b * CH + q * 16, 16)] = (
                            rowg[jj2, pl.ds(q * 16, 16)])

                @pl.when(jj2 < 2)
                def _():
                    pltpu.sync_copy(row3.at[wid, t + 2],
                                    grow.at[pl.ds(b * CH, CH)])

                pltpu.async_copy(
                    u.at[grow.at[pl.ds(b * CH, CH)]], rows_g.at[b], gsem[b])

        return 0

    lax.fori_loop(0, NCHUNK // 2, pair, 0)

    pltpu.make_async_copy(rows_s.at[0], acc.at[scol.at[0]], ssem[0]).wait()
    pltpu.make_async_copy(rows_s.at[1], acc.at[scol.at[1]], ssem[1]).wait()
    plsc.subcore_barrier()

    def co(q, _):
        sl = pl.ds(sid * NSUB + q * CH, CH)
        pltpu.sync_copy(acc.at[sl], out.at[cid, sl])
        return 0

    lax.fori_loop(0, NSUB // CH, co, 0)


def _dinv_body(p_ref, o_ref):
    d = p_ref[0] + p_ref[1]
    o_ref[...] = jnp.where(d > 0, lax.rsqrt(jnp.where(d > 0, d, 1.0)), 0.0)


def _dinv_tc(degp):
    p = degp.reshape(2, 8, NPAD // 8)
    out = pl.pallas_call(
        _dinv_body,
        out_shape=jax.ShapeDtypeStruct((8, NPAD // 8), jnp.float32),
    )(p)
    return out.reshape(NPAD, 1)


_RB = 2048


def _bfpack(u):
    rb = u.shape[0]
    b32 = lax.bitcast_convert_type(u.reshape(rb, 4, 2, 16), jnp.int32)
    rnd = (b32 >> 16) & 1
    bf = lax.shift_right_logical(b32 + 0x7FFF + rnd, 16)
    ui = bf[:, :, 0, :] | (bf[:, :, 1, :] << 16)
    return ui.reshape(rb, 64)


def _scale_body(x_ref, d_ref, o_ref):
    o_ref[...] = _bfpack(x_ref[...] * d_ref[...])


def _scale_tc(x, d):
    return pl.pallas_call(
        _scale_body,
        grid=(NPAD // _RB,),
        in_specs=[
            pl.BlockSpec((_RB, D), lambda i: (i, 0)),
            pl.BlockSpec((_RB, 1), lambda i: (i, 0)),
        ],
        out_specs=pl.BlockSpec((_RB, D // 2), lambda i: (i, 0)),
        out_shape=jax.ShapeDtypeStruct((NPAD, D // 2), jnp.int32),
    )(x, d)


def _comb1_body(p_ref, d_ref, tx_ref, u_ref):
    d = d_ref[...]
    tx = -(d * (p_ref[0] + p_ref[1]))
    tx_ref[...] = tx
    u_ref[...] = _bfpack(d * tx)


def _comb1_tc(p, d):
    return pl.pallas_call(
        _comb1_body,
        grid=(NPAD // _RB,),
        in_specs=[
            pl.BlockSpec((2, _RB, D), lambda i: (0, i, 0)),
            pl.BlockSpec((_RB, 1), lambda i: (i, 0)),
        ],
        out_specs=[
            pl.BlockSpec((_RB, D), lambda i: (i, 0)),
            pl.BlockSpec((_RB, D // 2), lambda i: (i, 0)),
        ],
        out_shape=[
            jax.ShapeDtypeStruct((NPAD, D), jnp.float32),
            jax.ShapeDtypeStruct((NPAD, D // 2), jnp.int32),
        ],
    )(p, d)


def _comb2_body(p_ref, d_ref, prev_ref, tx_ref, u_ref):
    d = d_ref[...]
    tx = -2.0 * (d * (p_ref[0] + p_ref[1])) - prev_ref[...]
    tx_ref[...] = tx
    u_ref[...] = _bfpack(d * tx)


def _comb2_tc(p, d, prev):
    return pl.pallas_call(
        _comb2_body,
        grid=(NPAD // _RB,),
        in_specs=[
            pl.BlockSpec((2, _RB, D), lambda i: (0, i, 0)),
            pl.BlockSpec((_RB, 1), lambda i: (i, 0)),
            pl.BlockSpec((_RB, D), lambda i: (i, 0)),
        ],
        out_specs=[
            pl.BlockSpec((_RB, D), lambda i: (i, 0)),
            pl.BlockSpec((_RB, D // 2), lambda i: (i, 0)),
        ],
        out_shape=[
            jax.ShapeDtypeStruct((NPAD, D), jnp.float32),
            jax.ShapeDtypeStruct((NPAD, D // 2), jnp.int32),
        ],
    )(p, d, prev)


_MB = 2000


def _mat_body(t0_ref, t1_ref, t2_ref, p3_ref, d_ref, w_ref, b_ref, o_ref):
    t3 = -2.0 * (d_ref[...] * (p3_ref[0] + p3_ref[1])) - t1_ref[...]
    xcat = jnp.concatenate(
        [t0_ref[...], t1_ref[...], t2_ref[...], t3], axis=1)
    acc = jnp.dot(xcat, w_ref[...], preferred_element_type=jnp.float32)
    o_ref[...] = acc + b_ref[...]


def _mat_tc(t0, t1, t2, p3, d, wm, bm):
    return pl.pallas_call(
        _mat_body,
        grid=(N // _MB,),
        in_specs=[
            pl.BlockSpec((_MB, D), lambda i: (i, 0)),
            pl.BlockSpec((_MB, D), lambda i: (i, 0)),
            pl.BlockSpec((_MB, D), lambda i: (i, 0)),
            pl.BlockSpec((2, _MB, D), lambda i: (0, i, 0)),
            pl.BlockSpec((_MB, 1), lambda i: (i, 0)),
            pl.BlockSpec((4 * D, 300), lambda i: (0, 0)),
            pl.BlockSpec((1, 300), lambda i: (0, 0)),
        ],
        out_specs=pl.BlockSpec((_MB, 300), lambda i: (i, 0)),
        out_shape=jax.ShapeDtypeStruct((N, 300), jnp.float32),
    )(t0, t1, t2, p3, d, wm, bm)


def kernel(x, edge_index, edge_weight, W1_0, W1_1, b1, W2_0, W2_1, W2_2, b2,
           W3_0, W3_1, W3_2, W3_3, b3):
    pad = NCHUNK * CH - EPW
    row3 = jnp.pad(edge_index[0].reshape(NW, EPW), ((0, 0), (0, pad))
                   ).reshape(NW, NCHUNK, CH)
    col3 = jnp.pad(edge_index[1].reshape(NW, EPW), ((0, 0), (0, pad))
                   ).reshape(NW, NCHUNK, CH)
    w3 = jnp.pad(edge_weight.reshape(NW, EPW), ((0, 0), (0, pad))
                 ).reshape(NW, NCHUNK, CH)
    xp = jnp.pad(x, ((0, NPAD - N), (0, 0)))

    degp, wm3 = _deg(row3, col3, w3)
    d = _dinv_tc(degp)

    u0 = _scale_tc(xp, d)
    p1 = _prop(u0, row3, col3, wm3)
    tx1, u1 = _comb1_tc(p1, d)
    p2 = _prop(u1, row3, col3, wm3)
    tx2, u2 = _comb2_tc(p2, d, xp)
    p3 = _prop(u2, row3, col3, wm3)

    z = jnp.zeros((100, D), jnp.float32)
    wmat = jnp.concatenate([
        jnp.concatenate([W1_0, W2_0, W3_0], axis=0),
        jnp.concatenate([W1_1, W2_1, W3_1], axis=0),
        jnp.concatenate([z, W2_2, W3_2], axis=0),
        jnp.concatenate([z, z, W3_3], axis=0),
    ], axis=1).T
    bm = jnp.concatenate([b1, b2, b3]).reshape(1, 300)
    return _mat_tc(xp, tx1, tx2, p3[:, :N], d, wmat, bm)


# speedup vs baseline: 1.1284x; 1.1284x over previous
import functools

import jax
import jax.numpy as jnp
from jax import lax
from jax.experimental import pallas as pl
from jax.experimental.pallas import tpu as pltpu
from jax.experimental.pallas import tpu_sc as plsc

N = 10000
E = 320000
D = 128
NPAD = 10240
NW = 32
EPW = E // NW
CH = 80
NCHUNK = 128
NSUB = NPAD // 16

_mesh = plsc.VectorSubcoreMesh(core_axis_name="c", subcore_axis_name="s")


def _worker_id():
    return lax.axis_index("s") * 2 + lax.axis_index("c")


@functools.partial(
    pl.kernel,
    out_type=(
        jax.ShapeDtypeStruct((2, NPAD), jnp.float32),
        jax.ShapeDtypeStruct((NW, NCHUNK, CH), jnp.float32),
    ),
    mesh=_mesh,
    scratch_types=[
        pltpu.VMEM((NCHUNK, CH), jnp.int32),
        pltpu.VMEM((NCHUNK, CH), jnp.int32),
        pltpu.VMEM((NCHUNK, CH), jnp.float32),
        pltpu.VMEM((NCHUNK, CH), jnp.float32),
        pltpu.VMEM((NSUB,), jnp.float32),
        pltpu.VMEM_SHARED((NPAD,), jnp.float32),
    ],
)
def _deg(row3, col3, w3, out, wm_out, rowv, colv, wv, wmv, zbuf, dacc):
    cid = lax.axis_index("c")
    sid = lax.axis_index("s")
    wid = _worker_id()
    pltpu.sync_copy(row3.at[wid], rowv)
    pltpu.sync_copy(col3.at[wid], colv)
    pltpu.sync_copy(w3.at[wid], wv)

    def zb(k, _):
        zbuf[pl.ds(k * 16, 16)] = jnp.zeros((16,), jnp.float32)
        return 0

    lax.fori_loop(0, NSUB // 16, zb, 0)
    pltpu.sync_copy(zbuf, dacc.at[pl.ds(sid * NSUB, NSUB)])
    plsc.subcore_barrier()

    def cm(t, _):
        j = t // (CH // 16)
        k = t % (CH // 16)
        sl = pl.ds(k * 16, 16)
        r = rowv[j, sl]
        c = colv[j, sl]
        w = wv[j, sl]
        wmv[j, sl] = jnp.where(r != c, w, 0.0)
        return 0

    lax.fori_loop(0, NCHUNK * (CH // 16), cm, 0)
    pltpu.sync_copy(wmv, wm_out.at[wid])

    def sc(j, _):
        pltpu.sync_copy(wmv.at[j], dacc.at[rowv.at[j]], add=True)
        return 0

    lax.fori_loop(0, NCHUNK, sc, 0)
    plsc.subcore_barrier()
    pltpu.sync_copy(dacc.at[pl.ds(sid * NSUB, NSUB)],
                    out.at[cid, pl.ds(sid * NSUB, NSUB)])


GB = 8
NGRP = NCHUNK // GB


@functools.partial(
    pl.kernel,
    out_type=jax.ShapeDtypeStruct((2, NPAD, D), jnp.float32),
    mesh=_mesh,
    scratch_types=[
        pltpu.VMEM((GB, CH), jnp.int32),
        pltpu.VMEM((GB, CH), jnp.int32),
        pltpu.VMEM((GB, CH), jnp.float32),
        pltpu.VMEM((2, CH, D // 2), jnp.int32),
        pltpu.VMEM((2, CH, D), jnp.float32),
        pltpu.VMEM((2 * CH,), jnp.int32),
        pltpu.VMEM((2, CH), jnp.int32),
        pltpu.VMEM_SHARED((NPAD, D), jnp.float32),
        pltpu.SemaphoreType.DMA,
        pltpu.SemaphoreType.DMA,
        pltpu.SemaphoreType.DMA,
        pltpu.SemaphoreType.DMA,
    ],
    compiler_params=pltpu.CompilerParams(needs_layout_passes=False,
                                         use_tc_tiling_on_sc=False),
)
def _prop(u, row3, col3, wm3, out, rowg, colg, wg, rows_g, rows_s,
          grow, scol, acc, gsem0, gsem1, ssem0, ssem1):
    cid = lax.axis_index("c")
    sid = lax.axis_index("s")
    wid = _worker_id()
    gsem = (gsem0, gsem1)
    ssem = (ssem0, ssem1)

    def zb(t, _):
        i = t // (D // 16)
        k = t % (D // 16)
        rows_s[0, i, pl.ds(k * 16, 16)] = jnp.zeros((16,), jnp.float32)
        return 0

    lax.fori_loop(0, CH * (D // 16), zb, 0)

    pltpu.sync_copy(row3.at[wid, pl.ds(0, GB)], rowg)
    pltpu.sync_copy(col3.at[wid, pl.ds(0, GB)], colg)
    pltpu.sync_copy(wm3.at[wid, pl.ds(0, GB)], wg)
    for b in range(2):
        for q in range(CH // 16):
            grow[pl.ds(b * CH + q * 16, 16)] = rowg[b, pl.ds(q * 16, 16)]
    pltpu.async_copy(u.at[grow.at[pl.ds(0, CH)]], rows_g.at[0], gsem[0])
    pltpu.async_copy(u.at[grow.at[pl.ds(CH, CH)]], rows_g.at[1], gsem[1])

    def zc(q, _):
        pltpu.sync_copy(rows_s.at[0], acc.at[pl.ds(sid * NSUB + q * CH, CH)])
        return 0

    lax.fori_loop(0, NSUB // CH, zc, 0)
    plsc.subcore_barrier()

    def pair(s, _):
        for b in range(2):
            t = 2 * s + b
            jj = lax.rem(t, GB)

            if b == 0:
                @pl.when(jnp.logical_and(lax.rem(s, GB // 2) == 0, s > 0))
                def _():
                    sl_g = pl.ds((2 * s // GB) * GB, GB)
                    pltpu.sync_copy(row3.at[wid, sl_g], rowg)
                    pltpu.sync_copy(col3.at[wid, sl_g], colg)
                    pltpu.sync_copy(wm3.at[wid, sl_g], wg)

            pltpu.make_async_copy(
                u.at[grow.at[pl.ds(b * CH, CH)]], rows_g.at[b], gsem[b]).wait()

            @pl.when(s > 0)
            def _():
                pltpu.make_async_copy(
                    rows_s.at[b], acc.at[scol.at[b]], ssem[b]).wait()

            for q in range(CH // 16):
                scol[b, pl.ds(q * 16, 16)] = colg[jj, pl.ds(q * 16, 16)]

            def srow(q, _):
                wvec = wg[jj, pl.ds(q * 16, 16)]
                base = q * 16
                for l in range(16):
                    sc = wvec[l]
                    for k in range(D // 32):
                        v = rows_g[b, base + l, pl.ds(k * 16, 16)]
                        lo = plsc.bitcast(v << 16, jnp.float32)
                        hi = plsc.bitcast(v & jnp.int32(-65536), jnp.float32)
                        rows_s[b, base + l, pl.ds(k * 32, 16)] = lo * sc
                        rows_s[b, base + l, pl.ds(k * 32 + 16, 16)] = hi * sc
                return 0

            lax.fori_loop(0, CH // 16, srow, 0)

            pltpu.async_copy(
                rows_s.at[b], acc.at[scol.at[b]], ssem[b], add=True)

            @pl.when(t < NCHUNK - 2)
            def _():
                jj2 = lax.rem(t + 2, GB)

                @pl.when(jj2 >= 2)
                def _():
                    for q in range(CH // 16):
                        grow[pl.ds(b * CH + q * 16, 16)] = (
                            rowg[jj2, pl.ds(q * 16, 16)])

                @pl.when(jj2 < 2)
                def _():
                    pltpu.sync_copy(row3.at[wid, t + 2],
                                    grow.at[pl.ds(b * CH, CH)])

                pltpu.async_copy(
                    u.at[grow.at[pl.ds(b * CH, CH)]], rows_g.at[b], gsem[b])

        return 0

    lax.fori_loop(0, NCHUNK // 2, pair, 0)

    pltpu.make_async_copy(rows_s.at[0], acc.at[scol.at[0]], ssem[0]).wait()
    pltpu.make_async_copy(rows_s.at[1], acc.at[scol.at[1]], ssem[1]).wait()
    plsc.subcore_barrier()

    def co(q, _):
        sl = pl.ds(sid * NSUB + q * CH, CH)
        pltpu.sync_copy(acc.at[sl], out.at[cid, sl])
        return 0

    lax.fori_loop(0, NSUB // CH, co, 0)


def _dinv_body(p_ref, o_ref):
    d = p_ref[0] + p_ref[1]
    o_ref[...] = jnp.where(d > 0, lax.rsqrt(jnp.where(d > 0, d, 1.0)), 0.0)


def _dinv_tc(degp):
    p = degp.reshape(2, 8, NPAD // 8)
    out = pl.pallas_call(
        _dinv_body,
        out_shape=jax.ShapeDtypeStruct((8, NPAD // 8), jnp.float32),
    )(p)
    return out.reshape(NPAD, 1)


_RB = 2048


def _pack_u(u):
    ub = u.astype(jnp.bfloat16)
    u4 = ub.reshape(NPAD, 4, 32)
    us = jnp.stack([u4[:, :, :16], u4[:, :, 16:]], axis=-1)
    return lax.bitcast_convert_type(us.reshape(NPAD, 64, 2), jnp.int32)


def _scale_body(x_ref, d_ref, o_ref):
    o_ref[...] = x_ref[...] * d_ref[...]


def _scale_tc(x, d):
    return pl.pallas_call(
        _scale_body,
        grid=(NPAD // _RB,),
        in_specs=[
            pl.BlockSpec((_RB, D), lambda i: (i, 0)),
            pl.BlockSpec((_RB, 1), lambda i: (i, 0)),
        ],
        out_specs=pl.BlockSpec((_RB, D), lambda i: (i, 0)),
        out_shape=jax.ShapeDtypeStruct((NPAD, D), jnp.float32),
    )(x, d)


def _comb1_body(p_ref, d_ref, tx_ref, u_ref):
    d = d_ref[...]
    tx = -(d * (p_ref[0] + p_ref[1]))
    tx_ref[...] = tx
    u_ref[...] = d * tx


def _comb1_tc(p, d):
    return pl.pallas_call(
        _comb1_body,
        grid=(NPAD // _RB,),
        in_specs=[
            pl.BlockSpec((2, _RB, D), lambda i: (0, i, 0)),
            pl.BlockSpec((_RB, 1), lambda i: (i, 0)),
        ],
        out_specs=[
            pl.BlockSpec((_RB, D), lambda i: (i, 0)),
            pl.BlockSpec((_RB, D), lambda i: (i, 0)),
        ],
        out_shape=[
            jax.ShapeDtypeStruct((NPAD, D), jnp.float32),
            jax.ShapeDtypeStruct((NPAD, D), jnp.float32),
        ],
    )(p, d)


def _comb2_body(p_ref, d_ref, prev_ref, tx_ref, u_ref):
    d = d_ref[...]
    tx = -2.0 * (d * (p_ref[0] + p_ref[1])) - prev_ref[...]
    tx_ref[...] = tx
    u_ref[...] = d * tx


def _comb2_tc(p, d, prev):
    return pl.pallas_call(
        _comb2_body,
        grid=(NPAD // _RB,),
        in_specs=[
            pl.BlockSpec((2, _RB, D), lambda i: (0, i, 0)),
            pl.BlockSpec((_RB, 1), lambda i: (i, 0)),
            pl.BlockSpec((_RB, D), lambda i: (i, 0)),
        ],
        out_specs=[
            pl.BlockSpec((_RB, D), lambda i: (i, 0)),
            pl.BlockSpec((_RB, D), lambda i: (i, 0)),
        ],
        out_shape=[
            jax.ShapeDtypeStruct((NPAD, D), jnp.float32),
            jax.ShapeDtypeStruct((NPAD, D), jnp.float32),
        ],
    )(p, d, prev)


_MB = 2000


def _mat_body(t0_ref, t1_ref, t2_ref, p3_ref, d_ref, w_ref, b_ref, o_ref):
    t3 = -2.0 * (d_ref[...] * (p3_ref[0] + p3_ref[1])) - t1_ref[...]
    xcat = jnp.concatenate(
        [t0_ref[...], t1_ref[...], t2_ref[...], t3], axis=1)
    acc = jnp.dot(xcat, w_ref[...], preferred_element_type=jnp.float32)
    o_ref[...] = acc + b_ref[...]


def _mat_tc(t0, t1, t2, p3, d, wm, bm):
    return pl.pallas_call(
        _mat_body,
        grid=(N // _MB,),
        in_specs=[
            pl.BlockSpec((_MB, D), lambda i: (i, 0)),
            pl.BlockSpec((_MB, D), lambda i: (i, 0)),
            pl.BlockSpec((_MB, D), lambda i: (i, 0)),
            pl.BlockSpec((2, _MB, D), lambda i: (0, i, 0)),
            pl.BlockSpec((_MB, 1), lambda i: (i, 0)),
            pl.BlockSpec((4 * D, 300), lambda i: (0, 0)),
            pl.BlockSpec((1, 300), lambda i: (0, 0)),
        ],
        out_specs=pl.BlockSpec((_MB, 300), lambda i: (i, 0)),
        out_shape=jax.ShapeDtypeStruct((N, 300), jnp.float32),
    )(t0, t1, t2, p3, d, wm, bm)


def kernel(x, edge_index, edge_weight, W1_0, W1_1, b1, W2_0, W2_1, W2_2, b2,
           W3_0, W3_1, W3_2, W3_3, b3):
    pad = NCHUNK * CH - EPW
    row3 = jnp.pad(edge_index[0].reshape(NW, EPW), ((0, 0), (0, pad))
                   ).reshape(NW, NCHUNK, CH)
    col3 = jnp.pad(edge_index[1].reshape(NW, EPW), ((0, 0), (0, pad))
                   ).reshape(NW, NCHUNK, CH)
    w3 = jnp.pad(edge_weight.reshape(NW, EPW), ((0, 0), (0, pad))
                 ).reshape(NW, NCHUNK, CH)
    xp = jnp.pad(x, ((0, NPAD - N), (0, 0)))

    degp, wm3 = _deg(row3, col3, w3)
    d = _dinv_tc(degp)

    u0 = _scale_tc(xp, d)
    p1 = _prop(_pack_u(u0), row3, col3, wm3)
    tx1, u1 = _comb1_tc(p1, d)
    p2 = _prop(_pack_u(u1), row3, col3, wm3)
    tx2, u2 = _comb2_tc(p2, d, xp)
    p3 = _prop(_pack_u(u2), row3, col3, wm3)

    z = jnp.zeros((100, D), jnp.float32)
    wmat = jnp.concatenate([
        jnp.concatenate([W1_0, W2_0, W3_0], axis=0),
        jnp.concatenate([W1_1, W2_1, W3_1], axis=0),
        jnp.concatenate([z, W2_2, W3_2], axis=0),
        jnp.concatenate([z, z, W3_3], axis=0),
    ], axis=1).T
    bm = jnp.concatenate([b1, b2, b3]).reshape(1, 300)
    return _mat_tc(xp, tx1, tx2, p3[:, :N], d, wmat, bm)
